# baseline (speedup 1.0000x reference)
import functools

import jax
import jax.numpy as jnp
from jax import lax
from jax.experimental import pallas as pl
from jax.experimental.pallas import tpu as pltpu
from jax.experimental.pallas import tpu_sc as plsc

_NC, _NS = 2, 16
_NW = _NC * _NS
_V = 1000000
_D = 32
_B = 16384 * 26
_L = 16

_CB = 128
_NBLK = 7812
_KA = _NBLK // _NW


@functools.partial(
    pl.kernel,
    out_type=jax.ShapeDtypeStruct((_V * _D,), jnp.float32),
    mesh=plsc.VectorSubcoreMesh(core_axis_name="c", subcore_axis_name="s"),
    compiler_params=pltpu.CompilerParams(
        use_tc_tiling_on_sc=True, needs_layout_passes=False),
    scratch_types=[
        pltpu.VMEM((_D, _CB), jnp.float32),
        pltpu.VMEM((_D, _CB), jnp.float32),
        pltpu.VMEM((_CB * _D,), jnp.float32),
        pltpu.VMEM((_CB * _D,), jnp.float32),
        pltpu.SemaphoreType.DMA,
        pltpu.SemaphoreType.DMA,
        pltpu.SemaphoreType.DMA,
        pltpu.SemaphoreType.DMA,
    ],
)
def _format_kernel(table_t, table_l, in0, in1, tout0, tout1,
                   si0, si1, so0, so1):
    wid = lax.axis_index("s") * _NC + lax.axis_index("c")

    def issue_in(c, buf, sem):
        pltpu.async_copy(table_t.at[:, pl.ds(c * _CB, _CB)], buf, sem)

    iota32 = lax.iota(jnp.int32, _L) * _D

    def transpose(buf, tout, width):
        @plsc.parallel_loop(0, width // _L, unroll=4)
        def _(jg):
            ov = iota32 + jg * (_L * _D)
            for d in range(_D):
                v = buf[d, pl.ds(jg * _L, _L)]
                plsc.store_scatter(tout, [ov + d], v)

    def issue_out(c, tout, sem):
        pltpu.async_copy(tout, table_l.at[pl.ds(c * (_CB * _D), _CB * _D)], sem)

    issue_in(wid, in0, si0)

    def body(k, carry):
        c0 = wid + _NW * (2 * k)
        c1 = c0 + _NW
        c2 = c1 + _NW

        @pl.when(c1 < _NBLK)
        def _():
            issue_in(c1, in1, si1)

        pltpu.make_async_copy(table_t.at[:, pl.ds(0, _CB)], in0, si0).wait()

        @pl.when(k > 0)
        def _():
            pltpu.make_async_copy(
                tout0, table_l.at[pl.ds(0, _CB * _D)], so0).wait()

        transpose(in0, tout0, _CB)
        issue_out(c0, tout0, so0)

        @pl.when(c2 < _NBLK)
        def _():
            issue_in(c2, in0, si0)

        @pl.when(c1 < _NBLK)
        def _():
            pltpu.make_async_copy(table_t.at[:, pl.ds(0, _CB)], in1, si1).wait()

            @pl.when(k > 0)
            def _():
                pltpu.make_async_copy(
                    tout1, table_l.at[pl.ds(0, _CB * _D)], so1).wait()

            transpose(in1, tout1, _CB)
            issue_out(c1, tout1, so1)

        return carry

    lax.fori_loop(0, _KA // 2, body, 0, unroll=False)
    pltpu.make_async_copy(tout0, table_l.at[pl.ds(0, _CB * _D)], so0).wait()
    pltpu.make_async_copy(tout1, table_l.at[pl.ds(0, _CB * _D)], so1).wait()

    @pl.when(wid < 4)
    def _():
        c = (_NBLK - 4) + wid
        pltpu.make_async_copy(table_t.at[:, pl.ds(0, _CB)], in0, si0).wait()
        transpose(in0, tout0, _CB)
        pltpu.async_copy(
            tout0, table_l.at[pl.ds(c * (_CB * _D), _CB * _D)], so0)
        pltpu.make_async_copy(tout0, table_l.at[pl.ds(0, _CB * _D)], so0).wait()

    @pl.when(wid == 4)
    def _():
        tail = wid * 0 + 999936
        pltpu.async_copy(table_t.at[:, pl.ds(tail, _CB)], in0, si0)
        pltpu.make_async_copy(table_t.at[:, pl.ds(0, _CB)], in0, si0).wait()
        transpose(in0, tout0, 64)
        pltpu.async_copy(
            tout0.at[pl.ds(0, 64 * _D)],
            table_l.at[pl.ds(999936 * _D, 64 * _D)], so0)
        pltpu.make_async_copy(
            tout0.at[pl.ds(0, 64 * _D)],
            table_l.at[pl.ds(0, 64 * _D)], so0).wait()


_BPW = _B // _NW
_CHUNK = 1664
_NCHUNK = _BPW // _CHUNK


@functools.partial(
    pl.kernel,
    out_type=jax.ShapeDtypeStruct((_B, _D), jnp.float32),
    mesh=plsc.VectorSubcoreMesh(core_axis_name="c", subcore_axis_name="s"),
    compiler_params=pltpu.CompilerParams(use_tc_tiling_on_sc=False),
    scratch_types=[
        pltpu.VMEM((_BPW,), jnp.int32),
        pltpu.VMEM((_CHUNK, _D), jnp.float32),
        pltpu.VMEM((_CHUNK, _D), jnp.float32),
        pltpu.SemaphoreType.DMA,
        pltpu.SemaphoreType.DMA,
    ],
)
def _gather_kernel(idx_hbm, table_hbm, out_hbm, idx_v, rows0, rows1, sem0, sem1):
    wid = lax.axis_index("s") * _NC + lax.axis_index("c")
    base = wid * _BPW
    pltpu.sync_copy(idx_hbm.at[pl.ds(base, _BPW)], idx_v)

    bufs = (rows0, rows1)
    sems = (sem0, sem1)
    handles = [None, None]
    handles[0] = pltpu.async_copy(
        table_hbm.at[idx_v.at[pl.ds(0, _CHUNK)]], rows0, sem0)
    for c in range(_NCHUNK):
        cur = c % 2
        handles[cur].wait()
        if c + 1 < _NCHUNK:
            nxt = (c + 1) % 2
            handles[nxt] = pltpu.async_copy(
                table_hbm.at[idx_v.at[pl.ds((c + 1) * _CHUNK, _CHUNK)]],
                bufs[nxt], sems[nxt])
        pltpu.sync_copy(bufs[cur], out_hbm.at[pl.ds(base + c * _CHUNK, _CHUNK)])


def kernel(indices, table):
    table_l = _format_kernel(table.T)
    flat = indices.reshape(-1)
    out = _gather_kernel(flat, table_l.reshape(_V, _D))
    return out.reshape(indices.shape[0], indices.shape[1], _D)


# speedup vs baseline: 1.0043x; 1.0043x over previous
import functools

import jax
import jax.numpy as jnp
from jax import lax
from jax.experimental import pallas as pl
from jax.experimental.pallas import tpu as pltpu
from jax.experimental.pallas import tpu_sc as plsc

_NC, _NS = 2, 16
_NW = _NC * _NS
_V = 1000000
_D = 32
_B = 16384 * 26
_L = 16

_CB = 128
_NBLK = 7812
_KA = _NBLK // _NW


@functools.partial(
    pl.kernel,
    out_type=jax.ShapeDtypeStruct((_V * _D,), jnp.float32),
    mesh=plsc.VectorSubcoreMesh(core_axis_name="c", subcore_axis_name="s"),
    compiler_params=pltpu.CompilerParams(
        use_tc_tiling_on_sc=True, needs_layout_passes=False),
    scratch_types=[
        pltpu.VMEM((_D, _CB), jnp.float32),
        pltpu.VMEM((_D, _CB), jnp.float32),
        pltpu.VMEM((_CB * _D,), jnp.float32),
        pltpu.VMEM((_CB * _D,), jnp.float32),
        pltpu.SemaphoreType.DMA,
        pltpu.SemaphoreType.DMA,
        pltpu.SemaphoreType.DMA,
        pltpu.SemaphoreType.DMA,
    ],
)
def _format_kernel(table_t, table_l, in0, in1, tout0, tout1,
                   si0, si1, so0, so1):
    wid = lax.axis_index("s") * _NC + lax.axis_index("c")

    def issue_in(c, buf, sem):
        pltpu.async_copy(table_t.at[:, pl.ds(c * _CB, _CB)], buf, sem)

    iota32 = lax.iota(jnp.int32, _L) * _D

    def transpose(buf, tout, width):
        @plsc.parallel_loop(0, width // _L, unroll=4)
        def _(jg):
            ov = iota32 + jg * (_L * _D)
            for d in range(_D):
                v = buf[d, pl.ds(jg * _L, _L)]
                plsc.store_scatter(tout, [ov + d], v)

    def issue_out(c, tout, sem):
        pltpu.async_copy(tout, table_l.at[pl.ds(c * (_CB * _D), _CB * _D)], sem)

    issue_in(wid, in0, si0)

    def body(k, carry):
        c0 = wid + _NW * (2 * k)
        c1 = c0 + _NW
        c2 = c1 + _NW

        @pl.when(c1 < _NBLK)
        def _():
            issue_in(c1, in1, si1)

        pltpu.make_async_copy(table_t.at[:, pl.ds(0, _CB)], in0, si0).wait()

        @pl.when(k > 0)
        def _():
            pltpu.make_async_copy(
                tout0, table_l.at[pl.ds(0, _CB * _D)], so0).wait()

        transpose(in0, tout0, _CB)
        issue_out(c0, tout0, so0)

        @pl.when(c2 < _NBLK)
        def _():
            issue_in(c2, in0, si0)

        @pl.when(c1 < _NBLK)
        def _():
            pltpu.make_async_copy(table_t.at[:, pl.ds(0, _CB)], in1, si1).wait()

            @pl.when(k > 0)
            def _():
                pltpu.make_async_copy(
                    tout1, table_l.at[pl.ds(0, _CB * _D)], so1).wait()

            transpose(in1, tout1, _CB)
            issue_out(c1, tout1, so1)

        return carry

    lax.fori_loop(0, _KA // 2, body, 0, unroll=False)
    pltpu.make_async_copy(tout0, table_l.at[pl.ds(0, _CB * _D)], so0).wait()
    pltpu.make_async_copy(tout1, table_l.at[pl.ds(0, _CB * _D)], so1).wait()

    @pl.when(wid < 4)
    def _():
        c = (_NBLK - 4) + wid
        pltpu.make_async_copy(table_t.at[:, pl.ds(0, _CB)], in0, si0).wait()
        transpose(in0, tout0, _CB)
        pltpu.async_copy(
            tout0, table_l.at[pl.ds(c * (_CB * _D), _CB * _D)], so0)
        pltpu.make_async_copy(tout0, table_l.at[pl.ds(0, _CB * _D)], so0).wait()

    @pl.when(wid == 4)
    def _():
        tail = wid * 0 + 999936
        pltpu.async_copy(table_t.at[:, pl.ds(tail, _CB)], in0, si0)
        pltpu.make_async_copy(table_t.at[:, pl.ds(0, _CB)], in0, si0).wait()
        transpose(in0, tout0, 64)
        pltpu.async_copy(
            tout0.at[pl.ds(0, 64 * _D)],
            table_l.at[pl.ds(999936 * _D, 64 * _D)], so0)
        pltpu.make_async_copy(
            tout0.at[pl.ds(0, 64 * _D)],
            table_l.at[pl.ds(0, 64 * _D)], so0).wait()


_NF = 26
_NB = 16384
_CK = 512
_TPF = _NB // _CK
_NTASK = _NF * _TPF
_TPW = _NTASK // _NW
_KB = _TPW // 2


@functools.partial(
    pl.kernel,
    out_type=jax.ShapeDtypeStruct((_NF, _D, _NB), jnp.float32),
    mesh=plsc.VectorSubcoreMesh(core_axis_name="c", subcore_axis_name="s"),
    compiler_params=pltpu.CompilerParams(
        use_tc_tiling_on_sc=False, needs_layout_passes=False),
    scratch_types=[
        pltpu.VMEM((_CK,), jnp.int32),
        pltpu.VMEM((_CK,), jnp.int32),
        pltpu.VMEM((_CK, _D), jnp.float32),
        pltpu.VMEM((_CK, _D), jnp.float32),
        pltpu.VMEM((_D, _CK), jnp.float32),
        pltpu.VMEM((_D, _CK), jnp.float32),
        pltpu.SemaphoreType.DMA,
        pltpu.SemaphoreType.DMA,
        pltpu.SemaphoreType.DMA,
        pltpu.SemaphoreType.DMA,
        pltpu.SemaphoreType.DMA,
        pltpu.SemaphoreType.DMA,
    ],
)
def _gather_kernel(idx_t, table_hbm, out_t,
                   idx0, idx1, rows0, rows1, ob0, ob1,
                   si0, si1, sg0, sg1, so0, so1):
    wid = lax.axis_index("s") * _NC + lax.axis_index("c")
    dlo = lax.iota(jnp.int32, _L)
    dhi = dlo + _L

    def task_fb(t):
        return t // _TPF, (t % _TPF) * _CK

    def start_task(t, idx_buf, isem, rows, gsem):
        f, b0 = task_fb(t)
        pltpu.async_copy(idx_t.at[f, pl.ds(b0, _CK)], idx_buf, isem)
        pltpu.make_async_copy(idx_t.at[0, pl.ds(0, _CK)], idx_buf, isem).wait()
        pltpu.async_copy(table_hbm.at[idx_buf], rows, gsem)

    def transpose(rows, ob):
        @plsc.parallel_loop(0, _CK, unroll=4)
        def _(j):
            jv = jnp.full((_L,), 0, jnp.int32) + j
            plsc.store_scatter(ob, [dlo, jv], rows[j, pl.ds(0, _L)])
            plsc.store_scatter(ob, [dhi, jv], rows[j, pl.ds(_L, _L)])

    def finish_task(t, rows, gsem, ob, osem, drain):
        f, b0 = task_fb(t)
        pltpu.make_async_copy(table_hbm.at[idx0], rows, gsem).wait()

        @pl.when(drain)
        def _():
            fp, bp = task_fb(t - 2 * _NW)
            pltpu.make_async_copy(
                ob, out_t.at[fp, :, pl.ds(bp, _CK)], osem).wait()

        transpose(rows, ob)
        pltpu.async_copy(ob, out_t.at[f, :, pl.ds(b0, _CK)], osem)

    start_task(wid, idx0, si0, rows0, sg0)

    def body(k, carry):
        t0 = wid + _NW * (2 * k)
        t1 = t0 + _NW
        t2 = t1 + _NW
        start_task(t1, idx1, si1, rows1, sg1)
        finish_task(t0, rows0, sg0, ob0, so0, k > 0)

        @pl.when(k < _KB - 1)
        def _():
            start_task(t2, idx0, si0, rows0, sg0)

        finish_task(t1, rows1, sg1, ob1, so1, k > 0)
        return carry

    lax.fori_loop(0, _KB, body, 0, unroll=False)
    f, b0 = task_fb(wid + _NW * (2 * (_KB - 1)))
    pltpu.make_async_copy(ob0, out_t.at[f, :, pl.ds(b0, _CK)], so0).wait()
    f, b0 = task_fb(wid + _NW * (2 * (_KB - 1) + 1))
    pltpu.make_async_copy(ob1, out_t.at[f, :, pl.ds(b0, _CK)], so1).wait()


def kernel(indices, table):
    table_l = _format_kernel(table.T)
    out_t = _gather_kernel(indices.T, table_l.reshape(_V, _D))
    return out_t.transpose(2, 0, 1)


# speedup vs baseline: 1.0503x; 1.0457x over previous
import functools

import jax
import jax.numpy as jnp
from jax import lax
from jax.experimental import pallas as pl
from jax.experimental.pallas import tpu as pltpu
from jax.experimental.pallas import tpu_sc as plsc

_NC, _NS = 2, 16
_NW = _NC * _NS
_B = 16384 * 26
_D = 32
_BPW = _B // _NW
_CHUNK = 1664
_NCHUNK = _BPW // _CHUNK


@functools.partial(
    pl.kernel,
    out_type=jax.ShapeDtypeStruct((_B, _D), jnp.float32),
    mesh=plsc.VectorSubcoreMesh(core_axis_name="c", subcore_axis_name="s"),
    compiler_params=pltpu.CompilerParams(use_tc_tiling_on_sc=False),
    scratch_types=[
        pltpu.VMEM((_BPW,), jnp.int32),
        pltpu.VMEM((_CHUNK, _D), jnp.float32),
        pltpu.VMEM((_CHUNK, _D), jnp.float32),
        pltpu.SemaphoreType.DMA,
        pltpu.SemaphoreType.DMA,
    ],
)
def _gather_kernel(idx_hbm, table_hbm, out_hbm, idx_v, rows0, rows1, sem0, sem1):
    wid = lax.axis_index("s") * _NC + lax.axis_index("c")
    base = wid * _BPW
    pltpu.sync_copy(idx_hbm.at[pl.ds(base, _BPW)], idx_v)

    bufs = (rows0, rows1)
    sems = (sem0, sem1)
    handles = [None, None]
    handles[0] = pltpu.async_copy(
        table_hbm.at[idx_v.at[pl.ds(0, _CHUNK)]], rows0, sem0)
    for c in range(_NCHUNK):
        cur = c % 2
        handles[cur].wait()
        if c + 1 < _NCHUNK:
            nxt = (c + 1) % 2
            handles[nxt] = pltpu.async_copy(
                table_hbm.at[idx_v.at[pl.ds((c + 1) * _CHUNK, _CHUNK)]],
                bufs[nxt], sems[nxt])
        pltpu.sync_copy(bufs[cur], out_hbm.at[pl.ds(base + c * _CHUNK, _CHUNK)])


def kernel(indices, table):
    flat = indices.reshape(-1)
    out = _gather_kernel(flat, table)
    return out.reshape(indices.shape[0], indices.shape[1], _D)
